# Initial kernel scaffold; baseline (speedup 1.0000x reference)
import jax
import jax.numpy as jnp
from jax.experimental import pallas as pl


def kernel(node_attr, edge_index, edge_attr, w1_0, b1_0, gamma_0, beta_0, w2_0, b2_0, root_0, bias_0, w1_1, b1_1, gamma_1, beta_1, w2_1, b2_1, root_1, bias_1):
    raise NotImplementedError("write your pallas kernel here")


# speedup vs baseline: 1.0549x; 1.0549x over previous
import functools

import jax
import jax.numpy as jnp
from jax import lax
from jax.experimental import pallas as pl
from jax.experimental.pallas import tpu as pltpu
from jax.experimental.pallas import tpu_sc as plsc

N_NODES = 10000
N_EDGES = 80000
EDGE_DIM = 16
EMBED = 32
EPS = 1e-5

NUM_CORES = 2
NUM_SUB = 16
NW = NUM_CORES * NUM_SUB
E_PAD = 81920
PER_W = E_PAD // NW
BATCH = 128
NB = PER_W // BATCH
N_PAD = 10240
ROWS_PER_SUB = N_PAD // NUM_SUB
T_MSG = 2048
T_NODE = 2000


def _sc_mesh():
    return plsc.VectorSubcoreMesh(
        core_axis_name="c", subcore_axis_name="s",
        num_cores=NUM_CORES, num_subcores=NUM_SUB)


@functools.cache
def _gather_fn(in_ch):
    def body(x_hbm, idx_hbm, out_hbm, idx_v, rows_v, sem):
        c = lax.axis_index("c")
        s = lax.axis_index("s")
        w = c * NUM_SUB + s
        pltpu.sync_copy(idx_hbm.at[w], idx_v)

        def step(j, carry):
            pltpu.async_copy(x_hbm.at[idx_v.at[j]], rows_v, sem).wait()
            pltpu.sync_copy(rows_v,
                            out_hbm.at[pl.ds(w * PER_W + j * BATCH, BATCH)])
            return carry

        lax.fori_loop(0, NB, step, 0)

    return pl.kernel(
        body,
        out_type=jax.ShapeDtypeStruct((E_PAD, in_ch), jnp.float32),
        mesh=_sc_mesh(),
        compiler_params=pltpu.CompilerParams(use_tc_tiling_on_sc=False),
        scratch_types=[
            pltpu.VMEM((NB, BATCH), jnp.int32),
            pltpu.VMEM((BATCH, in_ch), jnp.float32),
            pltpu.SemaphoreType.DMA,
        ])


@functools.cache
def _scatter_fn():
    def body(msg_hbm, idx_hbm, zeros_hbm, out_hbm, shared, idx_v, msg_v):
        c = lax.axis_index("c")
        s = lax.axis_index("s")
        w = c * NUM_SUB + s
        pltpu.sync_copy(zeros_hbm, shared.at[pl.ds(s * ROWS_PER_SUB, ROWS_PER_SUB)])
        plsc.subcore_barrier()
        pltpu.sync_copy(msg_hbm.at[pl.ds(w * PER_W, PER_W)], msg_v)
        pltpu.sync_copy(idx_hbm.at[w], idx_v)

        def step(j, carry):
            pltpu.sync_copy(msg_v.at[pl.ds(j * BATCH, BATCH)],
                            shared.at[idx_v.at[j]], add=True)
            return carry

        lax.fori_loop(0, NB, step, 0)
        plsc.subcore_barrier()
        pltpu.sync_copy(shared.at[pl.ds(s * ROWS_PER_SUB, ROWS_PER_SUB)],
                        out_hbm.at[c, pl.ds(s * ROWS_PER_SUB, ROWS_PER_SUB)])

    return pl.kernel(
        body,
        out_type=jax.ShapeDtypeStruct((NUM_CORES, N_PAD, EMBED), jnp.float32),
        mesh=_sc_mesh(),
        compiler_params=pltpu.CompilerParams(use_tc_tiling_on_sc=False),
        scratch_types=[
            pltpu.VMEM_SHARED((N_PAD, EMBED), jnp.float32),
            pltpu.VMEM((NB, BATCH), jnp.int32),
            pltpu.VMEM((PER_W, EMBED), jnp.float32),
        ])


def _stats_call(ea, w1t0, b10, g0, be0, w1t1, b11, g1, be1):
    TS = 8000
    nsteps = N_EDGES // TS

    def body(ea_ref, w1t0_ref, b10_ref, g0_ref, be0_ref,
             w1t1_ref, b11_ref, g1_ref, be1_ref, out_ref, m_acc, s_acc):
        i = pl.program_id(0)

        @pl.when(i == 0)
        def _():
            m_acc[...] = jnp.zeros_like(m_acc)
            s_acc[...] = jnp.zeros_like(s_acc)

        ea_t = ea_ref[...]
        m_acc[...] += jnp.sum(ea_t, axis=0, keepdims=True)
        s_acc[...] += lax.dot_general(ea_t, ea_t, (((0,), (0,)), ((), ())),
                                      preferred_element_type=jnp.float32)

        @pl.when(i == nsteps - 1)
        def _():
            mE = m_acc[...] / N_EDGES
            SE = s_acc[...] / N_EDGES
            rows = []
            for (w1t_r, b1_r, g_r, be_r) in (
                    (w1t0_ref, b10_ref, g0_ref, be0_ref),
                    (w1t1_ref, b11_ref, g1_ref, be1_ref)):
                w1t = w1t_r[...]
                b1 = b1_r[...]
                t = jnp.dot(mE, w1t, preferred_element_type=jnp.float32)
                mean = t + b1
                bq = jnp.dot(SE, w1t, preferred_element_type=jnp.float32)
                q = jnp.sum(w1t * bq, axis=0, keepdims=True)
                var = (q + 2.0 * b1 * t + b1 * b1) - mean * mean
                scale = g_r[...] * lax.rsqrt(var + EPS)
                shift = be_r[...] + scale * (b1 - mean)
                rows.append(scale)
                rows.append(shift)
            out_ref[...] = jnp.concatenate(rows, axis=0)

    small = lambda shape: pl.BlockSpec(shape, lambda i: (0, 0))
    return pl.pallas_call(
        body,
        grid=(nsteps,),
        in_specs=[pl.BlockSpec((TS, EDGE_DIM), lambda i: (i, 0)),
                  small((EDGE_DIM, EMBED)), small((1, EMBED)),
                  small((1, EMBED)), small((1, EMBED)),
                  small((EDGE_DIM, EMBED)), small((1, EMBED)),
                  small((1, EMBED)), small((1, EMBED))],
        out_specs=small((4, EMBED)),
        out_shape=jax.ShapeDtypeStruct((4, EMBED), jnp.float32),
        scratch_shapes=[pltpu.VMEM((1, EDGE_DIM), jnp.float32),
                        pltpu.VMEM((EDGE_DIM, EDGE_DIM), jnp.float32)],
        compiler_params=pltpu.CompilerParams(
            dimension_semantics=("arbitrary",)),
    )(ea, w1t0, b10, g0, be0, w1t1, b11, g1, be1)


@functools.cache
def _msg_fn(in_ch):
    grid = (E_PAD // T_MSG,)

    def body(ea_ref, xs_ref, w1t_ref, w2q_ref, b2r_ref, ss_ref, msg_ref):
        h0 = jnp.dot(ea_ref[...], w1t_ref[...],
                     preferred_element_type=jnp.float32)
        h = jnp.maximum(h0 * ss_ref[0:1, :] + ss_ref[1:2, :], 0.0)
        xs = xs_ref[...]
        t2 = jnp.dot(xs, w2q_ref[...],
                     preferred_element_type=jnp.float32)
        acc = jnp.dot(xs, b2r_ref[...],
                      preferred_element_type=jnp.float32)
        for k in range(EMBED):
            acc = acc + h[:, k:k + 1] * t2[:, k * EMBED:(k + 1) * EMBED]
        msg_ref[...] = acc

    small = lambda shape: pl.BlockSpec(shape, lambda i: (0, 0))
    return pl.pallas_call(
        body,
        grid=grid,
        in_specs=[pl.BlockSpec((T_MSG, EDGE_DIM), lambda i: (i, 0)),
                  pl.BlockSpec((T_MSG, in_ch), lambda i: (i, 0)),
                  small((EDGE_DIM, EMBED)),
                  small((in_ch, EMBED * EMBED)),
                  small((in_ch, EMBED)),
                  small((2, EMBED))],
        out_specs=pl.BlockSpec((T_MSG, EMBED), lambda i: (i, 0)),
        out_shape=jax.ShapeDtypeStruct((E_PAD, EMBED), jnp.float32),
        compiler_params=pltpu.CompilerParams(
            dimension_semantics=("arbitrary",)),
    )


@functools.cache
def _node_fn(in_ch):
    grid = (N_NODES // T_NODE,)

    def body(agg_ref, x_ref, root_ref, bias_ref, out_ref):
        a = agg_ref[0] + agg_ref[1]
        r = jnp.dot(x_ref[...], root_ref[...], preferred_element_type=jnp.float32)
        out_ref[...] = jnp.maximum(a + r + bias_ref[...], 0.0)

    small = lambda shape: pl.BlockSpec(shape, lambda *_: (0,) * len(shape))
    return pl.pallas_call(
        body,
        grid=grid,
        in_specs=[pl.BlockSpec((NUM_CORES, T_NODE, EMBED), lambda i: (0, i, 0)),
                  pl.BlockSpec((T_NODE, in_ch), lambda i: (i, 0)),
                  small((in_ch, EMBED)),
                  small((1, EMBED))],
        out_specs=pl.BlockSpec((T_NODE, EMBED), lambda i: (i, 0)),
        out_shape=jax.ShapeDtypeStruct((N_NODES, EMBED), jnp.float32),
        compiler_params=pltpu.CompilerParams(
            dimension_semantics=("arbitrary",)),
    )


def kernel(node_attr, edge_index, edge_attr,
           w1_0, b1_0, gamma_0, beta_0, w2_0, b2_0, root_0, bias_0,
           w1_1, b1_1, gamma_1, beta_1, w2_1, b2_1, root_1, bias_1):
    src = edge_index[0].astype(jnp.int32)
    dst = edge_index[1].astype(jnp.int32)
    src2d = jnp.pad(src, (0, E_PAD - N_EDGES)).reshape(NW, NB, BATCH)
    dst2d = jnp.pad(dst, (0, E_PAD - N_EDGES),
                    constant_values=N_NODES).reshape(NW, NB, BATCH)
    ea_pad = jnp.pad(edge_attr, ((0, E_PAD - N_EDGES), (0, 0)))
    zeros_blk = jnp.zeros((ROWS_PER_SUB, EMBED), jnp.float32)

    r2 = lambda v: v.reshape(1, EMBED)

    def prep(w2, b2, in_ch):
        w2q = (w2.reshape(in_ch, EMBED, EMBED)
               .transpose(0, 2, 1).reshape(in_ch, EMBED * EMBED))
        return w2q, b2.reshape(in_ch, EMBED)

    w1t0, w1t1 = w1_0.T, w1_1.T
    w2q0, b2r0 = prep(w2_0, b2_0, 64)
    w2q1, b2r1 = prep(w2_1, b2_1, EMBED)

    stats = _stats_call(edge_attr, w1t0, r2(b1_0), r2(gamma_0), r2(beta_0),
                        w1t1, r2(b1_1), r2(gamma_1), r2(beta_1))

    x = node_attr
    for (ss, w1t, w2q, b2r, root, bias, in_ch) in (
            (stats[0:2], w1t0, w2q0, b2r0, root_0, bias_0, 64),
            (stats[2:4], w1t1, w2q1, b2r1, root_1, bias_1, EMBED)):
        xs = _gather_fn(in_ch)(x, src2d)
        msg = _msg_fn(in_ch)(ea_pad, xs, w1t, w2q, b2r, ss)
        agg = _scatter_fn()(msg, dst2d, zeros_blk)
        x = _node_fn(in_ch)(agg, x, root, r2(bias))
    return x


# speedup vs baseline: 2.7100x; 2.5688x over previous
import functools

import jax
import jax.numpy as jnp
from jax import lax
from jax.experimental import pallas as pl
from jax.experimental.pallas import tpu as pltpu
from jax.experimental.pallas import tpu_sc as plsc

N_NODES = 10000
N_EDGES = 80000
EDGE_DIM = 16
EMBED = 32
EPS = 1e-5

NUM_CORES = 2
NUM_SUB = 16
NW = NUM_CORES * NUM_SUB
E_PAD = 81920
PER_W = E_PAD // NW
BATCH = 128
NB = PER_W // BATCH
N_PAD = 10240
ROWS_PER_SUB = N_PAD // NUM_SUB
T_MSG = 2048
T_NODE = 2000


def _sc_mesh():
    return plsc.VectorSubcoreMesh(
        core_axis_name="c", subcore_axis_name="s",
        num_cores=NUM_CORES, num_subcores=NUM_SUB)


@functools.cache
def _gather_fn(in_ch):
    def body(x_hbm, idx_hbm, out_hbm, idx_v, rows_v, sem):
        c = lax.axis_index("c")
        s = lax.axis_index("s")
        w = c * NUM_SUB + s
        pltpu.sync_copy(idx_hbm.at[w], idx_v)

        def step(j, carry):
            pltpu.async_copy(x_hbm.at[idx_v.at[j]], rows_v, sem).wait()
            pltpu.sync_copy(rows_v,
                            out_hbm.at[pl.ds(w * PER_W + j * BATCH, BATCH)])
            return carry

        lax.fori_loop(0, NB, step, 0)

    return pl.kernel(
        body,
        out_type=jax.ShapeDtypeStruct((E_PAD, in_ch), jnp.float32),
        mesh=_sc_mesh(),
        compiler_params=pltpu.CompilerParams(use_tc_tiling_on_sc=False),
        scratch_types=[
            pltpu.VMEM((NB, BATCH), jnp.int32),
            pltpu.VMEM((BATCH, in_ch), jnp.float32),
            pltpu.SemaphoreType.DMA,
        ])


@functools.cache
def _scatter_fn():
    def body(msg_hbm, idx_hbm, zeros_hbm, out_hbm, shared, idx_v, msg_v):
        c = lax.axis_index("c")
        s = lax.axis_index("s")
        w = c * NUM_SUB + s
        pltpu.sync_copy(zeros_hbm, shared.at[pl.ds(s * ROWS_PER_SUB, ROWS_PER_SUB)])
        plsc.subcore_barrier()
        pltpu.sync_copy(msg_hbm.at[pl.ds(w * PER_W, PER_W)], msg_v)
        pltpu.sync_copy(idx_hbm.at[w], idx_v)

        def step(j, carry):
            pltpu.sync_copy(msg_v.at[pl.ds(j * BATCH, BATCH)],
                            shared.at[idx_v.at[j]], add=True)
            return carry

        lax.fori_loop(0, NB, step, 0)
        plsc.subcore_barrier()
        pltpu.sync_copy(shared.at[pl.ds(s * ROWS_PER_SUB, ROWS_PER_SUB)],
                        out_hbm.at[c, pl.ds(s * ROWS_PER_SUB, ROWS_PER_SUB)])

    return pl.kernel(
        body,
        out_type=jax.ShapeDtypeStruct((NUM_CORES, N_PAD, EMBED), jnp.float32),
        mesh=_sc_mesh(),
        compiler_params=pltpu.CompilerParams(use_tc_tiling_on_sc=False),
        scratch_types=[
            pltpu.VMEM_SHARED((N_PAD, EMBED), jnp.float32),
            pltpu.VMEM((NB, BATCH), jnp.int32),
            pltpu.VMEM((PER_W, EMBED), jnp.float32),
        ])


def _stats_call(ea, w1t0, b10, g0, be0, w1t1, b11, g1, be1):
    TS = 8000
    nsteps = N_EDGES // TS

    def body(ea_ref, w1t0_ref, b10_ref, g0_ref, be0_ref,
             w1t1_ref, b11_ref, g1_ref, be1_ref, out_ref, m_acc, s_acc):
        i = pl.program_id(0)

        @pl.when(i == 0)
        def _():
            m_acc[...] = jnp.zeros_like(m_acc)
            s_acc[...] = jnp.zeros_like(s_acc)

        ea_t = ea_ref[...]
        m_acc[...] += jnp.sum(ea_t, axis=0, keepdims=True)
        s_acc[...] += lax.dot_general(ea_t, ea_t, (((0,), (0,)), ((), ())),
                                      preferred_element_type=jnp.float32)

        @pl.when(i == nsteps - 1)
        def _():
            mE = m_acc[...] / N_EDGES
            SE = s_acc[...] / N_EDGES
            rows = []
            for (w1t_r, b1_r, g_r, be_r) in (
                    (w1t0_ref, b10_ref, g0_ref, be0_ref),
                    (w1t1_ref, b11_ref, g1_ref, be1_ref)):
                w1t = w1t_r[...]
                b1 = b1_r[...]
                t = jnp.dot(mE, w1t, preferred_element_type=jnp.float32)
                mean = t + b1
                bq = jnp.dot(SE, w1t, preferred_element_type=jnp.float32)
                q = jnp.sum(w1t * bq, axis=0, keepdims=True)
                var = (q + 2.0 * b1 * t + b1 * b1) - mean * mean
                scale = g_r[...] * lax.rsqrt(var + EPS)
                shift = be_r[...] + scale * (b1 - mean)
                rows.append(scale)
                rows.append(shift)
            out_ref[...] = jnp.concatenate(rows, axis=0)

    small = lambda shape: pl.BlockSpec(shape, lambda i: (0, 0))
    return pl.pallas_call(
        body,
        grid=(nsteps,),
        in_specs=[pl.BlockSpec((TS, EDGE_DIM), lambda i: (i, 0)),
                  small((EDGE_DIM, EMBED)), small((1, EMBED)),
                  small((1, EMBED)), small((1, EMBED)),
                  small((EDGE_DIM, EMBED)), small((1, EMBED)),
                  small((1, EMBED)), small((1, EMBED))],
        out_specs=small((4, EMBED)),
        out_shape=jax.ShapeDtypeStruct((4, EMBED), jnp.float32),
        scratch_shapes=[pltpu.VMEM((1, EDGE_DIM), jnp.float32),
                        pltpu.VMEM((EDGE_DIM, EDGE_DIM), jnp.float32)],
        compiler_params=pltpu.CompilerParams(
            dimension_semantics=("arbitrary",)),
    )(ea, w1t0, b10, g0, be0, w1t1, b11, g1, be1)


@functools.cache
def _msg_fn(in_ch):
    grid = (E_PAD // T_MSG,)

    def body(ea_ref, xs_ref, w1tr_ref, w2q_ref, b2r_ref, ss_ref, msg_ref):
        h0 = jnp.dot(ea_ref[...], w1tr_ref[...],
                     preferred_element_type=jnp.float32)
        h_rep = jnp.maximum(h0 * ss_ref[0:1, :] + ss_ref[1:2, :], 0.0)
        xs = xs_ref[...]
        t2 = jnp.dot(xs, w2q_ref[...],
                     preferred_element_type=jnp.float32)
        p = h_rep * t2
        width = EMBED * EMBED
        while width > EMBED:
            width //= 2
            p = p[:, :width] + p[:, width:2 * width]
        msg_ref[...] = p + jnp.dot(xs, b2r_ref[...],
                                   preferred_element_type=jnp.float32)

    small = lambda shape: pl.BlockSpec(shape, lambda i: (0, 0))
    return pl.pallas_call(
        body,
        grid=grid,
        in_specs=[pl.BlockSpec((T_MSG, EDGE_DIM), lambda i: (i, 0)),
                  pl.BlockSpec((T_MSG, in_ch), lambda i: (i, 0)),
                  small((EDGE_DIM, EMBED * EMBED)),
                  small((in_ch, EMBED * EMBED)),
                  small((in_ch, EMBED)),
                  small((2, EMBED * EMBED))],
        out_specs=pl.BlockSpec((T_MSG, EMBED), lambda i: (i, 0)),
        out_shape=jax.ShapeDtypeStruct((E_PAD, EMBED), jnp.float32),
        compiler_params=pltpu.CompilerParams(
            dimension_semantics=("arbitrary",)),
    )


@functools.cache
def _node_fn(in_ch):
    grid = (N_NODES // T_NODE,)

    def body(agg_ref, x_ref, root_ref, bias_ref, out_ref):
        a = agg_ref[0] + agg_ref[1]
        r = jnp.dot(x_ref[...], root_ref[...], preferred_element_type=jnp.float32)
        out_ref[...] = jnp.maximum(a + r + bias_ref[...], 0.0)

    small = lambda shape: pl.BlockSpec(shape, lambda *_: (0,) * len(shape))
    return pl.pallas_call(
        body,
        grid=grid,
        in_specs=[pl.BlockSpec((NUM_CORES, T_NODE, EMBED), lambda i: (0, i, 0)),
                  pl.BlockSpec((T_NODE, in_ch), lambda i: (i, 0)),
                  small((in_ch, EMBED)),
                  small((1, EMBED))],
        out_specs=pl.BlockSpec((T_NODE, EMBED), lambda i: (i, 0)),
        out_shape=jax.ShapeDtypeStruct((N_NODES, EMBED), jnp.float32),
        compiler_params=pltpu.CompilerParams(
            dimension_semantics=("arbitrary",)),
    )


def kernel(node_attr, edge_index, edge_attr,
           w1_0, b1_0, gamma_0, beta_0, w2_0, b2_0, root_0, bias_0,
           w1_1, b1_1, gamma_1, beta_1, w2_1, b2_1, root_1, bias_1):
    src = edge_index[0].astype(jnp.int32)
    dst = edge_index[1].astype(jnp.int32)
    src2d = jnp.pad(src, (0, E_PAD - N_EDGES)).reshape(NW, NB, BATCH)
    dst2d = jnp.pad(dst, (0, E_PAD - N_EDGES),
                    constant_values=N_NODES).reshape(NW, NB, BATCH)
    ea_pad = jnp.pad(edge_attr, ((0, E_PAD - N_EDGES), (0, 0)))
    zeros_blk = jnp.zeros((ROWS_PER_SUB, EMBED), jnp.float32)

    r2 = lambda v: v.reshape(1, EMBED)

    def prep(w2, b2, in_ch):
        w2q = (w2.reshape(in_ch, EMBED, EMBED)
               .transpose(0, 2, 1).reshape(in_ch, EMBED * EMBED))
        return w2q, b2.reshape(in_ch, EMBED)

    w1t0, w1t1 = w1_0.T, w1_1.T
    w2q0, b2r0 = prep(w2_0, b2_0, 64)
    w2q1, b2r1 = prep(w2_1, b2_1, EMBED)

    stats = _stats_call(edge_attr, w1t0, r2(b1_0), r2(gamma_0), r2(beta_0),
                        w1t1, r2(b1_1), r2(gamma_1), r2(beta_1))

    rep = lambda v: jnp.repeat(v, EMBED, axis=1)

    x = node_attr
    for (ss, w1t, w2q, b2r, root, bias, in_ch) in (
            (stats[0:2], w1t0, w2q0, b2r0, root_0, bias_0, 64),
            (stats[2:4], w1t1, w2q1, b2r1, root_1, bias_1, EMBED)):
        xs = _gather_fn(in_ch)(x, src2d)
        msg = _msg_fn(in_ch)(ea_pad, xs, rep(w1t), w2q, b2r, rep(ss))
        agg = _scatter_fn()(msg, dst2d, zeros_blk)
        x = _node_fn(in_ch)(agg, x, root, r2(bias))
    return x


# speedup vs baseline: 2.7907x; 1.0298x over previous
import functools

import jax
import jax.numpy as jnp
from jax import lax
from jax.experimental import pallas as pl
from jax.experimental.pallas import tpu as pltpu
from jax.experimental.pallas import tpu_sc as plsc

N_NODES = 10000
N_EDGES = 80000
EDGE_DIM = 16
EMBED = 32
EPS = 1e-5

NUM_CORES = 2
NUM_SUB = 16
NW = NUM_CORES * NUM_SUB
E_PAD = 81920
PER_W = E_PAD // NW
BATCH = 128
NB = PER_W // BATCH
N_PAD = 10240
ROWS_PER_SUB = N_PAD // NUM_SUB
T_MSG = 2048
T_NODE = 2000


def _sc_mesh():
    return plsc.VectorSubcoreMesh(
        core_axis_name="c", subcore_axis_name="s",
        num_cores=NUM_CORES, num_subcores=NUM_SUB)


NSLOT = 4


@functools.cache
def _gather_fn(in_ch):
    def body(x_hbm, idx_hbm, out_hbm, idx_v, rows_v, *sems):
        gsems, wsems = sems[:NSLOT], sems[NSLOT:]
        c = lax.axis_index("c")
        s = lax.axis_index("s")
        w = c * NUM_SUB + s
        pltpu.sync_copy(idx_hbm.at[w], idx_v)

        def g_start(j):
            return pltpu.async_copy(
                x_hbm.at[idx_v.at[j]], rows_v.at[j % NSLOT], gsems[j % NSLOT])

        def w_start(j):
            return pltpu.async_copy(
                rows_v.at[j % NSLOT],
                out_hbm.at[pl.ds(w * PER_W + j * BATCH, BATCH)],
                wsems[j % NSLOT])

        gd = [None] * NB
        wd = [None] * NB
        for j in range(NB):
            if j >= NSLOT:
                wd[j - NSLOT].wait()
            gd[j] = g_start(j)
            k = j - 2
            if k >= 0:
                gd[k].wait()
                wd[k] = w_start(k)
        for k in range(NB - 2, NB):
            gd[k].wait()
            wd[k] = w_start(k)
        for k in range(NB - NSLOT, NB):
            wd[k].wait()

    return pl.kernel(
        body,
        out_type=jax.ShapeDtypeStruct((E_PAD, in_ch), jnp.float32),
        mesh=_sc_mesh(),
        compiler_params=pltpu.CompilerParams(use_tc_tiling_on_sc=False),
        scratch_types=[
            pltpu.VMEM((NB, BATCH), jnp.int32),
            pltpu.VMEM((NSLOT, BATCH, in_ch), jnp.float32),
        ] + [pltpu.SemaphoreType.DMA] * (2 * NSLOT))


@functools.cache
def _scatter_fn():
    def body(msg_hbm, idx_hbm, zeros_hbm, out_hbm, shared, idx_v, msg_v, ssem):
        c = lax.axis_index("c")
        s = lax.axis_index("s")
        w = c * NUM_SUB + s
        pltpu.sync_copy(zeros_hbm, shared.at[pl.ds(s * ROWS_PER_SUB, ROWS_PER_SUB)])
        plsc.subcore_barrier()
        pltpu.sync_copy(msg_hbm.at[pl.ds(w * PER_W, PER_W)], msg_v)
        pltpu.sync_copy(idx_hbm.at[w], idx_v)

        def step(j, carry):
            pltpu.async_copy(msg_v.at[pl.ds(j * BATCH, BATCH)],
                             shared.at[idx_v.at[j]], ssem, add=True).wait()
            return carry

        lax.fori_loop(0, NB, step, 0)
        plsc.subcore_barrier()
        pltpu.sync_copy(shared.at[pl.ds(s * ROWS_PER_SUB, ROWS_PER_SUB)],
                        out_hbm.at[c, pl.ds(s * ROWS_PER_SUB, ROWS_PER_SUB)])

    return pl.kernel(
        body,
        out_type=jax.ShapeDtypeStruct((NUM_CORES, N_PAD, EMBED), jnp.float32),
        mesh=_sc_mesh(),
        compiler_params=pltpu.CompilerParams(use_tc_tiling_on_sc=False),
        scratch_types=[
            pltpu.VMEM_SHARED((N_PAD, EMBED), jnp.float32),
            pltpu.VMEM((NB, BATCH), jnp.int32),
            pltpu.VMEM((PER_W, EMBED), jnp.float32),
            pltpu.SemaphoreType.DMA,
        ])


def _stats_call(ea, w1t0, b10, g0, be0, w1t1, b11, g1, be1):
    TS = 8000
    nsteps = N_EDGES // TS

    def body(ea_ref, w1t0_ref, b10_ref, g0_ref, be0_ref,
             w1t1_ref, b11_ref, g1_ref, be1_ref, out_ref, m_acc, s_acc):
        i = pl.program_id(0)

        @pl.when(i == 0)
        def _():
            m_acc[...] = jnp.zeros_like(m_acc)
            s_acc[...] = jnp.zeros_like(s_acc)

        ea_t = ea_ref[...]
        m_acc[...] += jnp.sum(ea_t, axis=0, keepdims=True)
        s_acc[...] += lax.dot_general(ea_t, ea_t, (((0,), (0,)), ((), ())),
                                      preferred_element_type=jnp.float32)

        @pl.when(i == nsteps - 1)
        def _():
            mE = m_acc[...] / N_EDGES
            SE = s_acc[...] / N_EDGES
            rows = []
            for (w1t_r, b1_r, g_r, be_r) in (
                    (w1t0_ref, b10_ref, g0_ref, be0_ref),
                    (w1t1_ref, b11_ref, g1_ref, be1_ref)):
                w1t = w1t_r[...]
                b1 = b1_r[...]
                t = jnp.dot(mE, w1t, preferred_element_type=jnp.float32)
                mean = t + b1
                bq = jnp.dot(SE, w1t, preferred_element_type=jnp.float32)
                q = jnp.sum(w1t * bq, axis=0, keepdims=True)
                var = (q + 2.0 * b1 * t + b1 * b1) - mean * mean
                scale = g_r[...] * lax.rsqrt(var + EPS)
                shift = be_r[...] + scale * (b1 - mean)
                rows.append(scale)
                rows.append(shift)
            out_ref[...] = jnp.concatenate(rows, axis=0)

    small = lambda shape: pl.BlockSpec(shape, lambda i: (0, 0))
    return pl.pallas_call(
        body,
        grid=(nsteps,),
        in_specs=[pl.BlockSpec((TS, EDGE_DIM), lambda i: (i, 0)),
                  small((EDGE_DIM, EMBED)), small((1, EMBED)),
                  small((1, EMBED)), small((1, EMBED)),
                  small((EDGE_DIM, EMBED)), small((1, EMBED)),
                  small((1, EMBED)), small((1, EMBED))],
        out_specs=small((4, EMBED)),
        out_shape=jax.ShapeDtypeStruct((4, EMBED), jnp.float32),
        scratch_shapes=[pltpu.VMEM((1, EDGE_DIM), jnp.float32),
                        pltpu.VMEM((EDGE_DIM, EDGE_DIM), jnp.float32)],
        compiler_params=pltpu.CompilerParams(
            dimension_semantics=("arbitrary",)),
    )(ea, w1t0, b10, g0, be0, w1t1, b11, g1, be1)


@functools.cache
def _msg_fn(in_ch):
    grid = (E_PAD // T_MSG,)

    def body(ea_ref, xs_ref, w1tr_ref, w2q_ref, b2r_ref, ss_ref, msg_ref):
        h0 = jnp.dot(ea_ref[...], w1tr_ref[...],
                     preferred_element_type=jnp.float32)
        h_rep = jnp.maximum(h0 * ss_ref[0:1, :] + ss_ref[1:2, :], 0.0)
        xs = xs_ref[...]
        t2 = jnp.dot(xs.astype(jnp.bfloat16), w2q_ref[...],
                     preferred_element_type=jnp.float32)
        p = h_rep * t2
        width = EMBED * EMBED
        while width > EMBED:
            width //= 2
            p = p[:, :width] + p[:, width:2 * width]
        msg_ref[...] = p + jnp.dot(xs, b2r_ref[...],
                                   preferred_element_type=jnp.float32)

    small = lambda shape: pl.BlockSpec(shape, lambda i: (0, 0))
    return pl.pallas_call(
        body,
        grid=grid,
        in_specs=[pl.BlockSpec((T_MSG, EDGE_DIM), lambda i: (i, 0)),
                  pl.BlockSpec((T_MSG, in_ch), lambda i: (i, 0)),
                  small((EDGE_DIM, EMBED * EMBED)),
                  small((in_ch, EMBED * EMBED)),
                  small((in_ch, EMBED)),
                  small((2, EMBED * EMBED))],
        out_specs=pl.BlockSpec((T_MSG, EMBED), lambda i: (i, 0)),
        out_shape=jax.ShapeDtypeStruct((E_PAD, EMBED), jnp.float32),
        compiler_params=pltpu.CompilerParams(
            dimension_semantics=("arbitrary",)),
    )


@functools.cache
def _node_fn(in_ch):
    grid = (N_NODES // T_NODE,)

    def body(agg_ref, x_ref, root_ref, bias_ref, out_ref):
        a = agg_ref[0] + agg_ref[1]
        r = jnp.dot(x_ref[...], root_ref[...], preferred_element_type=jnp.float32)
        out_ref[...] = jnp.maximum(a + r + bias_ref[...], 0.0)

    small = lambda shape: pl.BlockSpec(shape, lambda *_: (0,) * len(shape))
    return pl.pallas_call(
        body,
        grid=grid,
        in_specs=[pl.BlockSpec((NUM_CORES, T_NODE, EMBED), lambda i: (0, i, 0)),
                  pl.BlockSpec((T_NODE, in_ch), lambda i: (i, 0)),
                  small((in_ch, EMBED)),
                  small((1, EMBED))],
        out_specs=pl.BlockSpec((T_NODE, EMBED), lambda i: (i, 0)),
        out_shape=jax.ShapeDtypeStruct((N_NODES, EMBED), jnp.float32),
        compiler_params=pltpu.CompilerParams(
            dimension_semantics=("arbitrary",)),
    )


def kernel(node_attr, edge_index, edge_attr,
           w1_0, b1_0, gamma_0, beta_0, w2_0, b2_0, root_0, bias_0,
           w1_1, b1_1, gamma_1, beta_1, w2_1, b2_1, root_1, bias_1):
    src = edge_index[0].astype(jnp.int32)
    dst = edge_index[1].astype(jnp.int32)
    src2d = jnp.pad(src, (0, E_PAD - N_EDGES)).reshape(NW, NB, BATCH)
    dst2d = jnp.pad(dst, (0, E_PAD - N_EDGES),
                    constant_values=N_NODES).reshape(NW, NB, BATCH)
    ea_pad = jnp.pad(edge_attr, ((0, E_PAD - N_EDGES), (0, 0)))
    zeros_blk = jnp.zeros((ROWS_PER_SUB, EMBED), jnp.float32)

    r2 = lambda v: v.reshape(1, EMBED)

    def prep(w2, b2, in_ch):
        w2q = (w2.reshape(in_ch, EMBED, EMBED)
               .transpose(0, 2, 1).reshape(in_ch, EMBED * EMBED))
        return w2q.astype(jnp.bfloat16), b2.reshape(in_ch, EMBED)

    w1t0, w1t1 = w1_0.T, w1_1.T
    w2q0, b2r0 = prep(w2_0, b2_0, 64)
    w2q1, b2r1 = prep(w2_1, b2_1, EMBED)

    stats = _stats_call(edge_attr, w1t0, r2(b1_0), r2(gamma_0), r2(beta_0),
                        w1t1, r2(b1_1), r2(gamma_1), r2(beta_1))

    rep = lambda v: jnp.repeat(v, EMBED, axis=1)

    x = node_attr
    for (ss, w1t, w2q, b2r, root, bias, in_ch) in (
            (stats[0:2], w1t0, w2q0, b2r0, root_0, bias_0, 64),
            (stats[2:4], w1t1, w2q1, b2r1, root_1, bias_1, EMBED)):
        xs = _gather_fn(in_ch)(x, src2d).reshape(E_PAD, in_ch)
        msg = _msg_fn(in_ch)(ea_pad, xs, rep(w1t), w2q, b2r, rep(ss))
        agg = _scatter_fn()(msg, dst2d, zeros_blk)
        x = _node_fn(in_ch)(agg, x, root, r2(bias))
    return x


# speedup vs baseline: 2.8037x; 1.0047x over previous
import functools

import jax
import jax.numpy as jnp
from jax import lax
from jax.experimental import pallas as pl
from jax.experimental.pallas import tpu as pltpu
from jax.experimental.pallas import tpu_sc as plsc

N_NODES = 10000
N_EDGES = 80000
EDGE_DIM = 16
EMBED = 32
EPS = 1e-5

NUM_CORES = 2
NUM_SUB = 16
NW = NUM_CORES * NUM_SUB
E_PAD = 81920
PER_W = E_PAD // NW
BATCH = 128
NB = PER_W // BATCH
N_PAD = 10240
ROWS_PER_SUB = N_PAD // NUM_SUB
T_MSG = 4096
T_NODE = 2000


def _sc_mesh():
    return plsc.VectorSubcoreMesh(
        core_axis_name="c", subcore_axis_name="s",
        num_cores=NUM_CORES, num_subcores=NUM_SUB)


NSLOT = 4


@functools.cache
def _gather_fn(in_ch):
    def body(x_hbm, idx_hbm, out_hbm, idx_v, rows_v, *sems):
        gsems, wsems = sems[:NSLOT], sems[NSLOT:]
        c = lax.axis_index("c")
        s = lax.axis_index("s")
        w = c * NUM_SUB + s
        pltpu.sync_copy(idx_hbm.at[w], idx_v)

        def g_start(j):
            return pltpu.async_copy(
                x_hbm.at[idx_v.at[j]], rows_v.at[j % NSLOT], gsems[j % NSLOT])

        def w_start(j):
            return pltpu.async_copy(
                rows_v.at[j % NSLOT],
                out_hbm.at[pl.ds(w * PER_W + j * BATCH, BATCH)],
                wsems[j % NSLOT])

        gd = [None] * NB
        wd = [None] * NB
        for j in range(NB):
            if j >= NSLOT:
                wd[j - NSLOT].wait()
            gd[j] = g_start(j)
            k = j - 2
            if k >= 0:
                gd[k].wait()
                wd[k] = w_start(k)
        for k in range(NB - 2, NB):
            gd[k].wait()
            wd[k] = w_start(k)
        for k in range(NB - NSLOT, NB):
            wd[k].wait()

    return pl.kernel(
        body,
        out_type=jax.ShapeDtypeStruct((E_PAD, in_ch), jnp.float32),
        mesh=_sc_mesh(),
        compiler_params=pltpu.CompilerParams(use_tc_tiling_on_sc=False),
        scratch_types=[
            pltpu.VMEM((NB, BATCH), jnp.int32),
            pltpu.VMEM((NSLOT, BATCH, in_ch), jnp.float32),
        ] + [pltpu.SemaphoreType.DMA] * (2 * NSLOT))


@functools.cache
def _scatter_fn():
    def body(msg_hbm, idx_hbm, zeros_hbm, out_hbm, shared, idx_v, msg_v, ssem):
        c = lax.axis_index("c")
        s = lax.axis_index("s")
        w = c * NUM_SUB + s
        pltpu.sync_copy(zeros_hbm, shared.at[pl.ds(s * ROWS_PER_SUB, ROWS_PER_SUB)])
        plsc.subcore_barrier()
        pltpu.sync_copy(msg_hbm.at[pl.ds(w * PER_W, PER_W)], msg_v)
        pltpu.sync_copy(idx_hbm.at[w], idx_v)

        def step(j, carry):
            pltpu.async_copy(msg_v.at[pl.ds(j * BATCH, BATCH)],
                             shared.at[idx_v.at[j]], ssem, add=True).wait()
            return carry

        lax.fori_loop(0, NB, step, 0)
        plsc.subcore_barrier()
        pltpu.sync_copy(shared.at[pl.ds(s * ROWS_PER_SUB, ROWS_PER_SUB)],
                        out_hbm.at[c, pl.ds(s * ROWS_PER_SUB, ROWS_PER_SUB)])

    return pl.kernel(
        body,
        out_type=jax.ShapeDtypeStruct((NUM_CORES, N_PAD, EMBED), jnp.float32),
        mesh=_sc_mesh(),
        compiler_params=pltpu.CompilerParams(use_tc_tiling_on_sc=False),
        scratch_types=[
            pltpu.VMEM_SHARED((N_PAD, EMBED), jnp.float32),
            pltpu.VMEM((NB, BATCH), jnp.int32),
            pltpu.VMEM((PER_W, EMBED), jnp.float32),
            pltpu.SemaphoreType.DMA,
        ])


def _stats_call(ea, w1t0, b10, g0, be0, w1t1, b11, g1, be1):
    TS = 8000
    nsteps = N_EDGES // TS

    def body(ea_ref, w1t0_ref, b10_ref, g0_ref, be0_ref,
             w1t1_ref, b11_ref, g1_ref, be1_ref, out_ref, m_acc, s_acc):
        i = pl.program_id(0)

        @pl.when(i == 0)
        def _():
            m_acc[...] = jnp.zeros_like(m_acc)
            s_acc[...] = jnp.zeros_like(s_acc)

        ea_t = ea_ref[...]
        m_acc[...] += jnp.sum(ea_t, axis=0, keepdims=True)
        s_acc[...] += lax.dot_general(ea_t, ea_t, (((0,), (0,)), ((), ())),
                                      preferred_element_type=jnp.float32)

        @pl.when(i == nsteps - 1)
        def _():
            mE = m_acc[...] / N_EDGES
            SE = s_acc[...] / N_EDGES
            rows = []
            for (w1t_r, b1_r, g_r, be_r) in (
                    (w1t0_ref, b10_ref, g0_ref, be0_ref),
                    (w1t1_ref, b11_ref, g1_ref, be1_ref)):
                w1t = w1t_r[...]
                b1 = b1_r[...]
                t = jnp.dot(mE, w1t, preferred_element_type=jnp.float32)
                mean = t + b1
                bq = jnp.dot(SE, w1t, preferred_element_type=jnp.float32)
                q = jnp.sum(w1t * bq, axis=0, keepdims=True)
                var = (q + 2.0 * b1 * t + b1 * b1) - mean * mean
                scale = g_r[...] * lax.rsqrt(var + EPS)
                shift = be_r[...] + scale * (b1 - mean)
                rows.append(scale)
                rows.append(shift)
            out_ref[...] = jnp.concatenate(rows, axis=0)

    small = lambda shape: pl.BlockSpec(shape, lambda i: (0, 0))
    return pl.pallas_call(
        body,
        grid=(nsteps,),
        in_specs=[pl.BlockSpec((TS, EDGE_DIM), lambda i: (i, 0)),
                  small((EDGE_DIM, EMBED)), small((1, EMBED)),
                  small((1, EMBED)), small((1, EMBED)),
                  small((EDGE_DIM, EMBED)), small((1, EMBED)),
                  small((1, EMBED)), small((1, EMBED))],
        out_specs=small((4, EMBED)),
        out_shape=jax.ShapeDtypeStruct((4, EMBED), jnp.float32),
        scratch_shapes=[pltpu.VMEM((1, EDGE_DIM), jnp.float32),
                        pltpu.VMEM((EDGE_DIM, EDGE_DIM), jnp.float32)],
        compiler_params=pltpu.CompilerParams(
            dimension_semantics=("arbitrary",)),
    )(ea, w1t0, b10, g0, be0, w1t1, b11, g1, be1)


@functools.cache
def _msg_fn(in_ch):
    grid = (E_PAD // T_MSG,)

    def body(ea_ref, xs_ref, w1ts_ref, w2q_ref, b2r_ref, sh_ref, sum_ref,
             msg_ref):
        h0 = jnp.dot(ea_ref[...], w1ts_ref[...],
                     preferred_element_type=jnp.float32)
        h_rep = jnp.maximum(h0 + sh_ref[...], 0.0)
        xs = xs_ref[...]
        t2 = jnp.dot(xs.astype(jnp.bfloat16), w2q_ref[...],
                     preferred_element_type=jnp.float32)
        p = h_rep * t2
        width = EMBED * EMBED
        while width > EMBED:
            width //= 2
            p = p[:, :width] + p[:, width:2 * width]
        del sum_ref
        msg_ref[...] = p + jnp.dot(xs, b2r_ref[...],
                                   preferred_element_type=jnp.float32)

    small = lambda shape: pl.BlockSpec(shape, lambda i: (0, 0))
    return pl.pallas_call(
        body,
        grid=grid,
        in_specs=[pl.BlockSpec((T_MSG, EDGE_DIM), lambda i: (i, 0)),
                  pl.BlockSpec((T_MSG, in_ch), lambda i: (i, 0)),
                  small((EDGE_DIM, EMBED * EMBED)),
                  small((in_ch, EMBED * EMBED)),
                  small((in_ch, EMBED)),
                  small((1, EMBED * EMBED)),
                  small((EMBED * EMBED, EMBED))],
        out_specs=pl.BlockSpec((T_MSG, EMBED), lambda i: (i, 0)),
        out_shape=jax.ShapeDtypeStruct((E_PAD, EMBED), jnp.float32),
        compiler_params=pltpu.CompilerParams(
            dimension_semantics=("arbitrary",)),
    )


@functools.cache
def _node_fn(in_ch):
    grid = (N_NODES // T_NODE,)

    def body(agg_ref, x_ref, root_ref, bias_ref, out_ref):
        a = agg_ref[0] + agg_ref[1]
        r = jnp.dot(x_ref[...], root_ref[...], preferred_element_type=jnp.float32)
        out_ref[...] = jnp.maximum(a + r + bias_ref[...], 0.0)

    small = lambda shape: pl.BlockSpec(shape, lambda *_: (0,) * len(shape))
    return pl.pallas_call(
        body,
        grid=grid,
        in_specs=[pl.BlockSpec((NUM_CORES, T_NODE, EMBED), lambda i: (0, i, 0)),
                  pl.BlockSpec((T_NODE, in_ch), lambda i: (i, 0)),
                  small((in_ch, EMBED)),
                  small((1, EMBED))],
        out_specs=pl.BlockSpec((T_NODE, EMBED), lambda i: (i, 0)),
        out_shape=jax.ShapeDtypeStruct((N_NODES, EMBED), jnp.float32),
        compiler_params=pltpu.CompilerParams(
            dimension_semantics=("arbitrary",)),
    )


def kernel(node_attr, edge_index, edge_attr,
           w1_0, b1_0, gamma_0, beta_0, w2_0, b2_0, root_0, bias_0,
           w1_1, b1_1, gamma_1, beta_1, w2_1, b2_1, root_1, bias_1):
    src = edge_index[0].astype(jnp.int32)
    dst = edge_index[1].astype(jnp.int32)
    src2d = jnp.pad(src, (0, E_PAD - N_EDGES)).reshape(NW, NB, BATCH)
    dst2d = jnp.pad(dst, (0, E_PAD - N_EDGES),
                    constant_values=N_NODES).reshape(NW, NB, BATCH)
    ea_pad = jnp.pad(edge_attr, ((0, E_PAD - N_EDGES), (0, 0)))
    zeros_blk = jnp.zeros((ROWS_PER_SUB, EMBED), jnp.float32)

    r2 = lambda v: v.reshape(1, EMBED)

    def prep(w2, b2, in_ch):
        w2q = (w2.reshape(in_ch, EMBED, EMBED)
               .transpose(0, 2, 1).reshape(in_ch, EMBED * EMBED))
        return w2q.astype(jnp.bfloat16), b2.reshape(in_ch, EMBED)

    w1t0, w1t1 = w1_0.T, w1_1.T
    w2q0, b2r0 = prep(w2_0, b2_0, 64)
    w2q1, b2r1 = prep(w2_1, b2_1, EMBED)

    stats = _stats_call(edge_attr, w1t0, r2(b1_0), r2(gamma_0), r2(beta_0),
                        w1t1, r2(b1_1), r2(gamma_1), r2(beta_1))

    rep = lambda v: jnp.repeat(v, EMBED, axis=1)
    sum32 = jnp.tile(jnp.eye(EMBED, dtype=jnp.bfloat16), (EMBED, 1))

    x = node_attr
    for (ss, w1t, w2q, b2r, root, bias, in_ch) in (
            (stats[0:2], w1t0, w2q0, b2r0, root_0, bias_0, 64),
            (stats[2:4], w1t1, w2q1, b2r1, root_1, bias_1, EMBED)):
        scale_rep = rep(ss[0:1])
        shift_rep = rep(ss[1:2])
        w1ts = rep(w1t) * scale_rep
        xs = _gather_fn(in_ch)(x, src2d).reshape(E_PAD, in_ch)
        msg = _msg_fn(in_ch)(ea_pad, xs, w1ts, w2q, b2r, shift_rep, sum32)
        agg = _scatter_fn()(msg, dst2d, zeros_blk)
        x = _node_fn(in_ch)(agg, x, root, r2(bias))
    return x
